# baseline (speedup 1.0000x reference)
import functools

import jax
import jax.numpy as jnp
from jax import lax
from jax.experimental import pallas as pl
from jax.experimental.pallas import tpu as pltpu
from jax.experimental.pallas import tpu_sc as plsc

EMBED_DIM = 32
LANES = 16
NROWS = 1000000
CHUNK = 32768
SPLIT = 13 * CHUNK
SC_CHUNK = 512


def _matvec_body(ut_ref, mt_ref, w_ref, p_ref, q_ref):
    n = ut_ref.shape[1]
    p = jnp.full((n,), w_ref[2 * EMBED_DIM], jnp.float32)
    q = jnp.zeros((n,), jnp.float32)
    for d in range(EMBED_DIM):
        p = p + ut_ref[d] * w_ref[d]
        q = q + mt_ref[d] * w_ref[EMBED_DIM + d]
    p_ref[...] = p
    q_ref[...] = q


def _tc_matvec(utT, mtT, wb):
    grid = (NROWS - SPLIT + CHUNK - 1) // CHUNK
    off = SPLIT // CHUNK
    return pl.pallas_call(
        _matvec_body,
        grid=(grid,),
        in_specs=[
            pl.BlockSpec((EMBED_DIM, CHUNK), lambda g: (0, g + off)),
            pl.BlockSpec((EMBED_DIM, CHUNK), lambda g: (0, g + off)),
            pl.BlockSpec(memory_space=pltpu.SMEM),
        ],
        out_specs=[
            pl.BlockSpec((CHUNK,), lambda g: (g + off,)),
            pl.BlockSpec((CHUNK,), lambda g: (g + off,)),
        ],
        out_shape=[
            jax.ShapeDtypeStruct((NROWS,), jnp.float32),
            jax.ShapeDtypeStruct((NROWS,), jnp.float32),
        ],
        compiler_params=pltpu.CompilerParams(
            dimension_semantics=("arbitrary",)),
    )(utT, mtT, wb)


def _sc_matvec(utT, mtT, wb):
    info = plsc.get_sparse_core_info()
    nw = info.num_cores * info.num_subcores
    cols = SPLIT // nw
    nchunks = cols // SC_CHUNK

    mesh = plsc.VectorSubcoreMesh(core_axis_name="c", subcore_axis_name="s")

    @functools.partial(
        pl.kernel,
        mesh=mesh,
        compiler_params=pltpu.CompilerParams(
            needs_layout_passes=False, use_tc_tiling_on_sc=True),
        out_type=[
            jax.ShapeDtypeStruct((SPLIT,), jnp.float32),
            jax.ShapeDtypeStruct((SPLIT,), jnp.float32),
        ],
        scratch_types=[
            pltpu.VMEM((2, EMBED_DIM, SC_CHUNK), jnp.float32),
            pltpu.VMEM((2, EMBED_DIM, SC_CHUNK), jnp.float32),
            pltpu.VMEM((80,), jnp.float32),
            pltpu.VMEM((cols,), jnp.float32),
            pltpu.VMEM((cols,), jnp.float32),
            pltpu.SemaphoreType.DMA,
            pltpu.SemaphoreType.DMA,
        ],
    )
    def sc_kernel(ut_hbm, mt_hbm, wb_hbm, p_hbm, q_hbm,
                  ubuf, mbuf, w_v, pv, qv, sem0, sem1):
        wid = lax.axis_index("s") * info.num_cores + lax.axis_index("c")
        col0 = wid * cols
        pltpu.sync_copy(wb_hbm, w_v)
        w0 = w_v[pl.ds(0, LANES)]
        w1 = w_v[pl.ds(LANES, LANES)]
        w2 = w_v[pl.ds(2 * LANES, LANES)]
        w3 = w_v[pl.ds(3 * LANES, LANES)]
        bias = w_v[pl.ds(4 * LANES, LANES)][0]
        wu = [w0[i] for i in range(LANES)] + [w1[i] for i in range(LANES)]
        wm = [w2[i] for i in range(LANES)] + [w3[i] for i in range(LANES)]
        sems = [sem0, sem1]

        def fire(c, parity):
            src = pl.ds(col0 + c * SC_CHUNK, SC_CHUNK)
            pltpu.async_copy(ut_hbm.at[:, src], ubuf.at[parity], sems[parity])
            pltpu.async_copy(mt_hbm.at[:, src], mbuf.at[parity], sems[parity])

        def drain(parity):
            pltpu.make_async_copy(ut_hbm.at[:, pl.ds(0, SC_CHUNK)],
                                  ubuf.at[parity], sems[parity]).wait()
            pltpu.make_async_copy(mt_hbm.at[:, pl.ds(0, SC_CHUNK)],
                                  mbuf.at[parity], sems[parity]).wait()

        def compute(c, parity):
            for g in range(SC_CHUNK // LANES):
                sl = pl.ds(g * LANES, LANES)
                accp = jnp.zeros((LANES,), jnp.float32) + bias
                accq = jnp.zeros((LANES,), jnp.float32)
                for d in range(EMBED_DIM):
                    accp = accp + ubuf[parity, d, sl] * wu[d]
                    accq = accq + mbuf[parity, d, sl] * wm[d]
                dst = pl.ds(c * SC_CHUNK + g * LANES, LANES)
                pv[dst] = accp
                qv[dst] = accq

        fire(0, 0)

        def body(i, carry):
            c0 = i * 2
            fire(c0 + 1, 1)
            drain(0)
            compute(c0, 0)

            @pl.when(i < nchunks // 2 - 1)
            def _():
                fire(c0 + 2, 0)

            drain(1)
            compute(c0 + 1, 1)
            return carry

        lax.fori_loop(0, nchunks // 2, body, 0)
        pltpu.sync_copy(pv, p_hbm.at[pl.ds(col0, cols)])
        pltpu.sync_copy(qv, q_hbm.at[pl.ds(col0, cols)])

    return sc_kernel(utT, mtT, wb)


def _combine(user_id, movie_id, p_lo, q_lo, p_hi, q_hi):
    B = user_id.shape[0]
    info = plsc.get_sparse_core_info()
    nw = info.num_cores * info.num_subcores
    bw = B // nw

    mesh = plsc.VectorSubcoreMesh(core_axis_name="c", subcore_axis_name="s")

    @functools.partial(
        pl.kernel,
        mesh=mesh,
        compiler_params=pltpu.CompilerParams(
            needs_layout_passes=False, use_tc_tiling_on_sc=False),
        out_type=jax.ShapeDtypeStruct((B,), jnp.float32),
        scratch_types=[
            pltpu.VMEM((bw,), jnp.int32),
            pltpu.VMEM((bw,), jnp.int32),
            pltpu.VMEM((bw,), jnp.int32),
            pltpu.VMEM((bw,), jnp.int32),
            pltpu.VMEM((bw,), jnp.float32),
            pltpu.VMEM((bw,), jnp.float32),
            pltpu.VMEM((bw,), jnp.float32),
            pltpu.VMEM((bw,), jnp.float32),
            pltpu.VMEM((bw,), jnp.float32),
            pltpu.SemaphoreType.DMA,
            pltpu.SemaphoreType.DMA,
        ],
    )
    def sc_kernel(uid_hbm, mid_hbm, plo_hbm, qlo_hbm, phi_hbm, qhi_hbm,
                  out_hbm, uidx_v, midx_v, ulo_v, mlo_v,
                  pvl, qvl, pvh, qvh, out_v, sem_a, sem_b):
        wid = lax.axis_index("s") * info.num_cores + lax.axis_index("c")
        base = wid * bw
        pltpu.sync_copy(uid_hbm.at[pl.ds(base, bw)], uidx_v)
        pltpu.sync_copy(mid_hbm.at[pl.ds(base, bw)], midx_v)
        for g in range(bw // LANES):
            sl = pl.ds(g * LANES, LANES)
            ulo_v[sl] = jnp.minimum(uidx_v[sl], SPLIT - 1)
            mlo_v[sl] = jnp.minimum(midx_v[sl], SPLIT - 1)
        ca = pltpu.async_copy(plo_hbm.at[ulo_v], pvl, sem_a)
        cb = pltpu.async_copy(qlo_hbm.at[mlo_v], qvl, sem_b)
        cc = pltpu.async_copy(phi_hbm.at[uidx_v], pvh, sem_a)
        cd = pltpu.async_copy(qhi_hbm.at[midx_v], qvh, sem_b)
        ca.wait()
        cb.wait()
        cc.wait()
        cd.wait()
        for g in range(bw // LANES):
            sl = pl.ds(g * LANES, LANES)
            pg = jnp.where(uidx_v[sl] < SPLIT, pvl[sl], pvh[sl])
            qg = jnp.where(midx_v[sl] < SPLIT, qvl[sl], qvh[sl])
            out_v[sl] = pg + qg
        pltpu.sync_copy(out_v, out_hbm.at[pl.ds(base, bw)])

    return sc_kernel(user_id, movie_id, p_lo, q_lo, p_hi, q_hi)


def kernel(user_id, movie_id, user_table, movie_table, W, b):
    B = user_id.shape[0]
    utT = user_table.T
    mtT = movie_table.T
    wb = jnp.zeros((80,), jnp.float32).at[:2 * EMBED_DIM].set(W[:, 0]).at[2 * EMBED_DIM].set(b[0])

    p_hi, q_hi = _tc_matvec(utT, mtT, wb)
    p_lo, q_lo = _sc_matvec(utT, mtT, wb)
    out = _combine(user_id, movie_id, p_lo, q_lo, p_hi, q_hi)
    return out.reshape(B, 1)


# speedup vs baseline: 1.4569x; 1.4569x over previous
import functools

import jax
import jax.numpy as jnp
from jax import lax
from jax.experimental import pallas as pl
from jax.experimental.pallas import tpu as pltpu
from jax.experimental.pallas import tpu_sc as plsc

EMBED_DIM = 32
LANES = 16
NROWS = 1000000
CHUNK = 32768
SPLIT = 8 * CHUNK
SC_CHUNK = 1024


def _matvec_body(ut_ref, mt_ref, w_ref, p_ref, q_ref):
    n = ut_ref.shape[1]
    p = jnp.full((n,), w_ref[2 * EMBED_DIM], jnp.float32)
    q = jnp.zeros((n,), jnp.float32)
    for d in range(EMBED_DIM):
        p = p + ut_ref[d] * w_ref[d]
        q = q + mt_ref[d] * w_ref[EMBED_DIM + d]
    p_ref[...] = p
    q_ref[...] = q


def _tc_matvec(utT, mtT, wb):
    grid = (NROWS - SPLIT + CHUNK - 1) // CHUNK
    off = SPLIT // CHUNK
    return pl.pallas_call(
        _matvec_body,
        grid=(grid,),
        in_specs=[
            pl.BlockSpec((EMBED_DIM, CHUNK), lambda g: (0, g + off)),
            pl.BlockSpec((EMBED_DIM, CHUNK), lambda g: (0, g + off)),
            pl.BlockSpec(memory_space=pltpu.SMEM),
        ],
        out_specs=[
            pl.BlockSpec((CHUNK,), lambda g: (g + off,)),
            pl.BlockSpec((CHUNK,), lambda g: (g + off,)),
        ],
        out_shape=[
            jax.ShapeDtypeStruct((NROWS,), jnp.float32),
            jax.ShapeDtypeStruct((NROWS,), jnp.float32),
        ],
        compiler_params=pltpu.CompilerParams(
            dimension_semantics=("arbitrary",)),
    )(utT, mtT, wb)


def _sc_matvec(utT, mtT, wb):
    info = plsc.get_sparse_core_info()
    nw = info.num_cores * info.num_subcores
    cols = SPLIT // nw
    nchunks = cols // SC_CHUNK

    mesh = plsc.VectorSubcoreMesh(core_axis_name="c", subcore_axis_name="s")

    @functools.partial(
        pl.kernel,
        mesh=mesh,
        compiler_params=pltpu.CompilerParams(
            needs_layout_passes=False, use_tc_tiling_on_sc=True),
        out_type=[
            jax.ShapeDtypeStruct((SPLIT,), jnp.float32),
            jax.ShapeDtypeStruct((SPLIT,), jnp.float32),
        ],
        scratch_types=[
            pltpu.VMEM((2, EMBED_DIM, SC_CHUNK), jnp.float32),
            pltpu.VMEM((80,), jnp.float32),
            pltpu.VMEM((cols,), jnp.float32),
            pltpu.VMEM((cols,), jnp.float32),
            pltpu.SemaphoreType.DMA,
            pltpu.SemaphoreType.DMA,
        ],
    )
    def sc_kernel(ut_hbm, mt_hbm, wb_hbm, p_hbm, q_hbm,
                  buf, w_v, pv, qv, sem0, sem1):
        wid = lax.axis_index("s") * info.num_cores + lax.axis_index("c")
        col0 = wid * cols
        pltpu.sync_copy(wb_hbm, w_v)
        w0 = w_v[pl.ds(0, LANES)]
        w1 = w_v[pl.ds(LANES, LANES)]
        w2 = w_v[pl.ds(2 * LANES, LANES)]
        w3 = w_v[pl.ds(3 * LANES, LANES)]
        bias = w_v[pl.ds(4 * LANES, LANES)][0]
        wu = [w0[i] for i in range(LANES)] + [w1[i] for i in range(LANES)]
        wm = [w2[i] for i in range(LANES)] + [w3[i] for i in range(LANES)]
        sems = [sem0, sem1]

        def one_pass(tab_hbm, out_v, w, acc_init):
            def fire(c, parity):
                src = pl.ds(col0 + c * SC_CHUNK, SC_CHUNK)
                pltpu.async_copy(tab_hbm.at[:, src], buf.at[parity],
                                 sems[parity])

            def drain(parity):
                pltpu.make_async_copy(tab_hbm.at[:, pl.ds(0, SC_CHUNK)],
                                      buf.at[parity], sems[parity]).wait()

            def compute(c, parity):
                def grp(j, carry):
                    for k in range(4):
                        g = j * 4 + k
                        sl = pl.ds(g * LANES, LANES)
                        parts = [
                            jnp.zeros((LANES,), jnp.float32) + acc_init,
                            jnp.zeros((LANES,), jnp.float32),
                            jnp.zeros((LANES,), jnp.float32),
                            jnp.zeros((LANES,), jnp.float32),
                        ]
                        for d in range(EMBED_DIM):
                            parts[d % 4] = (parts[d % 4]
                                            + buf[parity, d, sl] * w[d])
                        dst = pl.ds(c * SC_CHUNK + g * LANES, LANES)
                        out_v[dst] = (parts[0] + parts[1]) + (parts[2] + parts[3])
                    return carry

                lax.fori_loop(0, SC_CHUNK // LANES // 4, grp, 0)

            fire(0, 0)

            def body(i, carry):
                c0 = i * 2
                fire(c0 + 1, 1)
                drain(0)
                compute(c0, 0)

                @pl.when(i < nchunks // 2 - 1)
                def _():
                    fire(c0 + 2, 0)

                drain(1)
                compute(c0 + 1, 1)
                return carry

            lax.fori_loop(0, nchunks // 2, body, 0)

        one_pass(ut_hbm, pv, wu, bias)
        one_pass(mt_hbm, qv, wm, 0.0)
        pltpu.sync_copy(pv, p_hbm.at[pl.ds(col0, cols)])
        pltpu.sync_copy(qv, q_hbm.at[pl.ds(col0, cols)])

    return sc_kernel(utT, mtT, wb)


def _combine(user_id, movie_id, p_lo, q_lo, p_hi, q_hi):
    B = user_id.shape[0]
    info = plsc.get_sparse_core_info()
    nw = info.num_cores * info.num_subcores
    bw = B // nw

    mesh = plsc.VectorSubcoreMesh(core_axis_name="c", subcore_axis_name="s")

    @functools.partial(
        pl.kernel,
        mesh=mesh,
        compiler_params=pltpu.CompilerParams(
            needs_layout_passes=False, use_tc_tiling_on_sc=False),
        out_type=jax.ShapeDtypeStruct((B,), jnp.float32),
        scratch_types=[
            pltpu.VMEM((bw,), jnp.int32),
            pltpu.VMEM((bw,), jnp.int32),
            pltpu.VMEM((bw,), jnp.int32),
            pltpu.VMEM((bw,), jnp.int32),
            pltpu.VMEM((bw,), jnp.float32),
            pltpu.VMEM((bw,), jnp.float32),
            pltpu.VMEM((bw,), jnp.float32),
            pltpu.VMEM((bw,), jnp.float32),
            pltpu.VMEM((bw,), jnp.float32),
            pltpu.SemaphoreType.DMA,
            pltpu.SemaphoreType.DMA,
        ],
    )
    def sc_kernel(uid_hbm, mid_hbm, plo_hbm, qlo_hbm, phi_hbm, qhi_hbm,
                  out_hbm, uidx_v, midx_v, ulo_v, mlo_v,
                  pvl, qvl, pvh, qvh, out_v, sem_a, sem_b):
        wid = lax.axis_index("s") * info.num_cores + lax.axis_index("c")
        base = wid * bw
        pltpu.sync_copy(uid_hbm.at[pl.ds(base, bw)], uidx_v)
        pltpu.sync_copy(mid_hbm.at[pl.ds(base, bw)], midx_v)
        for g in range(bw // LANES):
            sl = pl.ds(g * LANES, LANES)
            ulo_v[sl] = jnp.minimum(uidx_v[sl], SPLIT - 1)
            mlo_v[sl] = jnp.minimum(midx_v[sl], SPLIT - 1)
        ca = pltpu.async_copy(plo_hbm.at[ulo_v], pvl, sem_a)
        cb = pltpu.async_copy(qlo_hbm.at[mlo_v], qvl, sem_b)
        cc = pltpu.async_copy(phi_hbm.at[uidx_v], pvh, sem_a)
        cd = pltpu.async_copy(qhi_hbm.at[midx_v], qvh, sem_b)
        ca.wait()
        cb.wait()
        cc.wait()
        cd.wait()
        for g in range(bw // LANES):
            sl = pl.ds(g * LANES, LANES)
            pg = jnp.where(uidx_v[sl] < SPLIT, pvl[sl], pvh[sl])
            qg = jnp.where(midx_v[sl] < SPLIT, qvl[sl], qvh[sl])
            out_v[sl] = pg + qg
        pltpu.sync_copy(out_v, out_hbm.at[pl.ds(base, bw)])

    return sc_kernel(user_id, movie_id, p_lo, q_lo, p_hi, q_hi)


def kernel(user_id, movie_id, user_table, movie_table, W, b):
    B = user_id.shape[0]
    utT = user_table.T
    mtT = movie_table.T
    wb = jnp.zeros((80,), jnp.float32).at[:2 * EMBED_DIM].set(W[:, 0]).at[2 * EMBED_DIM].set(b[0])

    p_hi, q_hi = _tc_matvec(utT, mtT, wb)
    p_lo, q_lo = _sc_matvec(utT, mtT, wb)
    out = _combine(user_id, movie_id, p_lo, q_lo, p_hi, q_hi)
    return out.reshape(B, 1)


# speedup vs baseline: 2.3357x; 1.6032x over previous
import functools

import jax
import jax.numpy as jnp
from jax import lax
from jax.experimental import pallas as pl
from jax.experimental.pallas import tpu as pltpu
from jax.experimental.pallas import tpu_sc as plsc

EMBED_DIM = 32
LANES = 16
NROWS = 1000000
CHUNK = 32768


def _matvec_body(ut_ref, mt_ref, w_ref, p_ref, q_ref):
    p = jnp.full((CHUNK,), w_ref[2 * EMBED_DIM], jnp.float32)
    q = jnp.zeros((CHUNK,), jnp.float32)
    for d in range(EMBED_DIM):
        p = p + ut_ref[d] * w_ref[d]
        q = q + mt_ref[d] * w_ref[EMBED_DIM + d]
    p_ref[...] = p
    q_ref[...] = q


def _combine(user_id, movie_id, p, q):
    B = user_id.shape[0]
    info = plsc.get_sparse_core_info()
    nw = info.num_cores * info.num_subcores
    bw = B // nw

    mesh = plsc.VectorSubcoreMesh(core_axis_name="c", subcore_axis_name="s")

    @functools.partial(
        pl.kernel,
        mesh=mesh,
        compiler_params=pltpu.CompilerParams(
            needs_layout_passes=False, use_tc_tiling_on_sc=False),
        out_type=jax.ShapeDtypeStruct((B,), jnp.float32),
        scratch_types=[
            pltpu.VMEM((bw,), jnp.int32),
            pltpu.VMEM((bw,), jnp.int32),
            pltpu.VMEM((bw,), jnp.float32),
            pltpu.VMEM((bw,), jnp.float32),
            pltpu.VMEM((bw,), jnp.float32),
            pltpu.SemaphoreType.DMA,
            pltpu.SemaphoreType.DMA,
        ],
    )
    def sc_kernel(uid_hbm, mid_hbm, p_hbm, q_hbm, out_hbm,
                  uidx_v, midx_v, pv, qv, out_v, sem_p, sem_q):
        wid = lax.axis_index("s") * info.num_cores + lax.axis_index("c")
        base = wid * bw
        pltpu.sync_copy(uid_hbm.at[pl.ds(base, bw)], uidx_v)
        pltpu.sync_copy(mid_hbm.at[pl.ds(base, bw)], midx_v)
        cp = pltpu.async_copy(p_hbm.at[uidx_v], pv, sem_p)
        cq = pltpu.async_copy(q_hbm.at[midx_v], qv, sem_q)
        cp.wait()
        cq.wait()
        for g in range(bw // LANES):
            sl = pl.ds(g * LANES, LANES)
            out_v[sl] = pv[sl] + qv[sl]
        pltpu.sync_copy(out_v, out_hbm.at[pl.ds(base, bw)])

    return sc_kernel(user_id, movie_id, p, q)


def kernel(user_id, movie_id, user_table, movie_table, W, b):
    B = user_id.shape[0]
    utT = user_table.T
    mtT = movie_table.T
    wb = jnp.zeros((80,), jnp.float32).at[:2 * EMBED_DIM].set(W[:, 0]).at[2 * EMBED_DIM].set(b[0])

    grid = (NROWS + CHUNK - 1) // CHUNK
    p, q = pl.pallas_call(
        _matvec_body,
        grid=(grid,),
        in_specs=[
            pl.BlockSpec((EMBED_DIM, CHUNK), lambda g: (0, g)),
            pl.BlockSpec((EMBED_DIM, CHUNK), lambda g: (0, g)),
            pl.BlockSpec(memory_space=pltpu.SMEM),
        ],
        out_specs=[
            pl.BlockSpec((CHUNK,), lambda g: (g,)),
            pl.BlockSpec((CHUNK,), lambda g: (g,)),
        ],
        out_shape=[
            jax.ShapeDtypeStruct((NROWS,), jnp.float32),
            jax.ShapeDtypeStruct((NROWS,), jnp.float32),
        ],
        compiler_params=pltpu.CompilerParams(
            dimension_semantics=("arbitrary",)),
    )(utT, mtT, wb)

    out = _combine(user_id, movie_id, p, q)
    return out.reshape(B, 1)
